# baseline (speedup 1.0000x reference)
import functools

import jax
import jax.numpy as jnp
from jax.experimental import pallas as pl
from jax.experimental.pallas import tpu as pltpu

_TOP_K = 8
_ALPHA = 0.001


def _gate_kernel(x_ref, w_ref, idx_ref, wgt_ref, aux_ref,
                 cnt_acc, ssum_acc, *, nblocks, blocks_per_batch,
                 num_batches, seq_len, num_experts):
    i = pl.program_id(0)

    @pl.when(i == 0)
    def _init():
        cnt_acc[...] = jnp.zeros_like(cnt_acc)
        ssum_acc[...] = jnp.zeros_like(ssum_acc)

    logits = jax.lax.dot_general(
        w_ref[...], x_ref[...],
        dimension_numbers=(((1,), (1,)), ((), ())),
        preferred_element_type=jnp.float32)

    t = logits.shape[1]
    c_w = 128
    eidf = jax.lax.broadcasted_iota(
        jnp.int32, (num_experts, c_w), 0).astype(jnp.float32)

    cnt_c = jnp.zeros((num_experts, c_w), jnp.float32)
    ssum_c = jnp.zeros((num_experts, c_w), jnp.float32)
    for c in range(t // c_w):
        lg = logits[:, c * c_w:(c + 1) * c_w]
        m = jnp.max(lg, axis=0, keepdims=True)
        ex = jnp.exp(lg - m)
        denom = jnp.sum(ex, axis=0, keepdims=True)
        sc = ex * (1.0 / denom)
        ssum_c += sc

        work = sc
        wgt_rows = []
        idx_rows = []
        for _ in range(_TOP_K):
            mk = jnp.max(work, axis=0, keepdims=True)
            is_max = work == mk
            idxk = jnp.min(jnp.where(is_max, eidf, float(num_experts)),
                           axis=0, keepdims=True)
            sel = eidf == idxk
            work = jnp.where(sel, -1.0, work)
            wgt_rows.append(mk)
            idx_rows.append(idxk)
        wgt_ref[:, c * c_w:(c + 1) * c_w] = jnp.concatenate(wgt_rows, 0)
        idx_ref[:, c * c_w:(c + 1) * c_w] = jnp.concatenate(
            idx_rows, 0).astype(jnp.int32)
        cnt_c += (work < 0).astype(jnp.float32)

    sel_cnt = jnp.sum(cnt_c, axis=1, keepdims=True)
    s_sum = jnp.sum(ssum_c, axis=1, keepdims=True)

    b = i // blocks_per_batch
    bhot = (jax.lax.broadcasted_iota(jnp.int32, (1, num_batches), 1)
            == b).astype(jnp.float32)
    cnt_acc[...] += sel_cnt * bhot
    ssum_acc[...] += s_sum * bhot

    @pl.when(i == nblocks - 1)
    def _finalize():
        ce = cnt_acc[...] * (num_experts / (seq_len * _TOP_K))
        mean_scores = ssum_acc[...] * (1.0 / seq_len)
        aux = (jnp.sum(ce * mean_scores) / num_batches) * _ALPHA
        aux_ref[...] = jnp.full((1, 1), aux, dtype=jnp.float32)


@jax.jit
def kernel(x, W):
    bsz, seq_len, dim = x.shape
    num_experts = W.shape[0]
    tokens = bsz * seq_len
    hidden = x.reshape(tokens, dim)

    block_t = 4096
    nblocks = tokens // block_t
    blocks_per_batch = seq_len // block_t

    kfn = functools.partial(
        _gate_kernel,
        nblocks=nblocks,
        blocks_per_batch=blocks_per_batch,
        num_batches=bsz,
        seq_len=seq_len,
        num_experts=num_experts,
    )

    idx_t, wgt_t, aux = pl.pallas_call(
        kfn,
        grid=(nblocks,),
        in_specs=[
            pl.BlockSpec((block_t, dim), lambda i: (i, 0)),
            pl.BlockSpec((num_experts, dim), lambda i: (0, 0)),
        ],
        out_specs=[
            pl.BlockSpec((_TOP_K, block_t), lambda i: (0, i)),
            pl.BlockSpec((_TOP_K, block_t), lambda i: (0, i)),
            pl.BlockSpec((1, 1), lambda i: (0, 0)),
        ],
        out_shape=[
            jax.ShapeDtypeStruct((_TOP_K, tokens), jnp.int32),
            jax.ShapeDtypeStruct((_TOP_K, tokens), jnp.float32),
            jax.ShapeDtypeStruct((1, 1), jnp.float32),
        ],
        scratch_shapes=[
            pltpu.VMEM((num_experts, bsz), jnp.float32),
            pltpu.VMEM((num_experts, bsz), jnp.float32),
        ],
    )(hidden, W)

    return idx_t.T, wgt_t.T, aux[0, 0]


# speedup vs baseline: 1.0409x; 1.0409x over previous
import functools

import jax
import jax.numpy as jnp
from jax.experimental import pallas as pl
from jax.experimental.pallas import tpu as pltpu

_TOP_K = 8
_ALPHA = 0.001


def _gate_kernel(x_ref, w_ref, idx_ref, wgt_ref, aux_ref,
                 cnt_acc, ssum_acc, *, nblocks, blocks_per_batch,
                 num_batches, seq_len, num_experts):
    i = pl.program_id(0)

    @pl.when(i == 0)
    def _init():
        cnt_acc[...] = jnp.zeros_like(cnt_acc)
        ssum_acc[...] = jnp.zeros_like(ssum_acc)

    logits = jax.lax.dot_general(
        w_ref[...], x_ref[...],
        dimension_numbers=(((1,), (1,)), ((), ())),
        preferred_element_type=jnp.float32)

    m = jnp.max(logits, axis=0, keepdims=True)
    e = jnp.exp(logits - m)
    denom = jnp.sum(e, axis=0, keepdims=True)
    scores = e * (1.0 / denom)

    t = scores.shape[1]
    eidi = jax.lax.broadcasted_iota(jnp.int32, (num_experts, t), 0)
    pw = jax.lax.bitcast_convert_type(
        (127 - eidi) << 23, jnp.float32)
    ones_row = jnp.ones((1, num_experts), jnp.float32)

    work = scores
    wgt_rows = []
    idx_rows = []
    for _ in range(_TOP_K):
        mk = jnp.max(work, axis=0, keepdims=True)
        is_max = work == mk
        s = jax.lax.dot_general(
            ones_row, jnp.where(is_max, pw, 0.0),
            dimension_numbers=(((1,), (0,)), ((), ())),
            preferred_element_type=jnp.float32)
        idxk = 127 - jax.lax.shift_right_logical(
            jax.lax.bitcast_convert_type(s, jnp.int32), 23)
        sel = eidi == idxk
        work = jnp.where(sel, -1.0, work)
        wgt_rows.append(mk)
        idx_rows.append(idxk)
    wgt_ref[...] = jnp.concatenate(wgt_rows, axis=0)
    idx_ref[...] = jnp.concatenate(idx_rows, axis=0)

    sel_cnt = jnp.sum((work < 0).astype(jnp.float32), axis=1,
                      keepdims=True)
    s_sum = jnp.sum(scores, axis=1, keepdims=True)

    b = i // blocks_per_batch
    bhot = (jax.lax.broadcasted_iota(jnp.int32, (1, num_batches), 1)
            == b).astype(jnp.float32)
    cnt_acc[...] += sel_cnt * bhot
    ssum_acc[...] += s_sum * bhot

    @pl.when(i == nblocks - 1)
    def _finalize():
        ce = cnt_acc[...] * (num_experts / (seq_len * _TOP_K))
        mean_scores = ssum_acc[...] * (1.0 / seq_len)
        aux = (jnp.sum(ce * mean_scores) / num_batches) * _ALPHA
        aux_ref[...] = jnp.full((1, 1), aux, dtype=jnp.float32)


@jax.jit
def kernel(x, W):
    bsz, seq_len, dim = x.shape
    num_experts = W.shape[0]
    tokens = bsz * seq_len
    hidden = x.reshape(tokens, dim)

    block_t = 4096
    nblocks = tokens // block_t
    blocks_per_batch = seq_len // block_t

    kfn = functools.partial(
        _gate_kernel,
        nblocks=nblocks,
        blocks_per_batch=blocks_per_batch,
        num_batches=bsz,
        seq_len=seq_len,
        num_experts=num_experts,
    )

    idx_t, wgt_t, aux = pl.pallas_call(
        kfn,
        grid=(nblocks,),
        in_specs=[
            pl.BlockSpec((block_t, dim), lambda i: (i, 0)),
            pl.BlockSpec((num_experts, dim), lambda i: (0, 0)),
        ],
        out_specs=[
            pl.BlockSpec((_TOP_K, block_t), lambda i: (0, i)),
            pl.BlockSpec((_TOP_K, block_t), lambda i: (0, i)),
            pl.BlockSpec((1, 1), lambda i: (0, 0)),
        ],
        out_shape=[
            jax.ShapeDtypeStruct((_TOP_K, tokens), jnp.int32),
            jax.ShapeDtypeStruct((_TOP_K, tokens), jnp.float32),
            jax.ShapeDtypeStruct((1, 1), jnp.float32),
        ],
        scratch_shapes=[
            pltpu.VMEM((num_experts, bsz), jnp.float32),
            pltpu.VMEM((num_experts, bsz), jnp.float32),
        ],
    )(hidden, W)

    return idx_t.T, wgt_t.T, aux[0, 0]
